# baseline (speedup 1.0000x reference)
import functools

import jax
import jax.numpy as jnp
from jax import lax
from jax.experimental import pallas as pl
from jax.experimental.pallas import tpu as pltpu
from jax.experimental.pallas import tpu_sc as plsc

B, S = 64, 200
NUM_C3, NUM_D, DIM_V = 2048, 10000, 128
CHUNK = 40
NCHUNK = S // CHUNK
HIST = 256
F32 = jnp.float32
I32 = jnp.int32


def _sc_prep(table_flat, d_flat, c3_flat):
    info = plsc.get_sparse_core_info()
    nc, ns, nl = info.num_cores, info.num_subcores, info.num_lanes
    nw = nc * ns
    total = B * S
    per = total // nw
    rows_per = per // S
    assert per % nl == 0 and rows_per * S == per

    mesh = plsc.VectorSubcoreMesh(core_axis_name="c", subcore_axis_name="s")

    @functools.partial(
        pl.kernel,
        mesh=mesh,
        out_type=(
            jax.ShapeDtypeStruct((total,), F32),
            jax.ShapeDtypeStruct((total,), I32),
        ),
        scratch_types=[
            pltpu.VMEM((NUM_D,), F32),
            pltpu.VMEM((per,), I32),
            pltpu.VMEM((per,), I32),
            pltpu.VMEM((per,), F32),
            pltpu.VMEM((per,), I32),
            pltpu.VMEM((rows_per * NUM_C3,), I32),
        ],
        compiler_params=pltpu.CompilerParams(needs_layout_passes=False),
    )
    def prep_kernel(tab_hbm, d_hbm, c3_hbm, gam_hbm, prev_hbm,
                    tab_v, d_v, c3_v, gam_v, prev_v, ls_v):
        wid = lax.axis_index("s") * nc + lax.axis_index("c")
        base = wid * per
        pltpu.sync_copy(tab_hbm, tab_v)
        pltpu.sync_copy(d_hbm.at[pl.ds(base, per)], d_v)
        pltpu.sync_copy(c3_hbm.at[pl.ds(base, per)], c3_v)

        for i in range(per // nl):
            sl = pl.ds(i * nl, nl)
            gam_v[sl] = plsc.load_gather(tab_v, [d_v[sl]])

        neg1 = jnp.full((nl,), -1, I32)
        for j in range(rows_per * NUM_C3 // nl):
            ls_v[pl.ds(j * nl, nl)] = neg1

        @pl.loop(0, S)
        def _chain(t):
            tv = jnp.zeros((nl,), I32) + t
            for r in range(rows_per):
                trow = tv + r * S
                idx = plsc.load_gather(c3_v, [trow]) + r * NUM_C3
                prev = plsc.load_gather(ls_v, [idx])
                plsc.store_scatter(ls_v, [idx], tv)
                plsc.store_scatter(prev_v, [trow], prev)

        pltpu.sync_copy(gam_v, gam_hbm.at[pl.ds(base, per)])
        pltpu.sync_copy(prev_v, prev_hbm.at[pl.ds(base, per)])

    return prep_kernel(table_flat, d_flat, c3_flat)


def _tc_body(gamma_ref, r_ref, c3_ref, prev_ref,
             WihT_ref, WhhT_ref, R_ref, vd_ref, vc3_ref,
             bih_ref, bhh_ref,
             W1aT_ref, w1b_ref, b1a_ref, b1b_ref,
             W2aT_ref, w2b_ref, b2a_ref, b2b_ref,
             alpha_out_ref, h_out_ref, c3_out_ref,
             h_scr, alpha_scr, hist_scr, slots, sems):
    pid = pl.program_id(0)

    @pl.when(pid == 0)
    def _init():
        h_scr[...] = jnp.zeros_like(h_scr)
        alpha_scr[...] = jnp.zeros_like(alpha_scr)
        hist_scr[...] = jnp.zeros_like(hist_scr)
        slots[CHUNK - 1] = jnp.zeros((B, NUM_C3), F32)

    vd_row = vd_ref[...]
    vc3_row = vc3_ref[...]

    bih_row = bih_ref[...]
    bhh_row = bhh_ref[...]
    b1a_row = b1a_ref[...]
    b2a_row = b2a_ref[...]
    w1bT = w1b_ref[...]
    w2bT = w2b_ref[...]
    b1b = b1b_ref[...]
    b2b = b2b_ref[...]
    WihT = WihT_ref[...]
    WhhT = WhhT_ref[...]
    W1aT = W1aT_ref[...]
    W2aT = W2aT_ref[...]
    R0 = R_ref[0:1, :]
    R1 = R_ref[1:2, :]

    iota_c3 = lax.broadcasted_iota(I32, (B, NUM_C3), 1)
    iota_h = lax.broadcasted_iota(I32, (B, HIST), 1)

    h = h_scr[...]
    alpha = alpha_scr[...]
    hist = hist_scr[...]

    xs = []
    for k in range(CHUNK):
        g_col = gamma_ref[0, :, k:k + 1]
        r_is1 = r_ref[0, :, k:k + 1] == 1
        vr_t = jnp.where(r_is1, R1, R0)
        xs.append(jnp.concatenate([g_col * vd_row, vr_t], axis=1))
    x_all = jnp.concatenate(xs, axis=0)
    gi_all = jnp.dot(x_all, WihT, preferred_element_type=F32) + bih_row

    p2_all = jnp.dot(x_all[:, 0:DIM_V], W2aT[DIM_V:2 * DIM_V, :],
                     preferred_element_type=F32)
    p3_all = jnp.dot(x_all[:, DIM_V:], W2aT[2 * DIM_V:, :],
                     preferred_element_type=F32)

    hs = []
    for k in range(CHUNK):
        t = pid * CHUNK + k
        c_col = c3_ref[0, :, k:k + 1]
        p_col = prev_ref[0, :, k:k + 1]

        gi = gi_all[k * B:(k + 1) * B, :]
        gh = jnp.dot(h, WhhT, preferred_element_type=F32) + bhh_row
        rg = jax.nn.sigmoid(gi[:, 0:DIM_V] + gh[:, 0:DIM_V])
        zg = jax.nn.sigmoid(gi[:, DIM_V:2 * DIM_V] + gh[:, DIM_V:2 * DIM_V])
        ng = jnp.tanh(gi[:, 2 * DIM_V:] + rg * gh[:, 2 * DIM_V:])
        h = (1.0 - zg) * ng + zg * h
        h_out_ref[:, k, :] = h
        hs.append(h)

        bmask = iota_h == p_col
        beta3 = jnp.sum(jnp.where(bmask, hist, 0.0), axis=1, keepdims=True)
        p1 = jnp.dot(beta3 * vc3_row, W2aT[0:DIM_V, :],
                     preferred_element_type=F32)
        pre2 = ((p1 + p2_all[k * B:(k + 1) * B, :])
                + p3_all[k * B:(k + 1) * B, :]) + b2a_row
        hid2 = jax.nn.relu(pre2)
        new_c3 = jnp.dot(hid2, w2bT, preferred_element_type=F32) + b2b
        hist = jnp.where(iota_h == t, new_c3, hist)

        @pl.when(pid > 0)
        def _wait_slot():
            pltpu.make_async_copy(
                slots.at[k], c3_out_ref.at[:, t, :], sems.at[k]).wait()
        prev_state = slots[k - 1] if k > 0 else slots[CHUNK - 1]
        slots[k] = jnp.where(iota_c3 == c_col, new_c3, prev_state)
        pltpu.make_async_copy(
            slots.at[k], c3_out_ref.at[:, t, :], sems.at[k]).start()

    h_all = jnp.concatenate(hs, axis=0)
    hid = jax.nn.relu(jnp.dot(h_all, W1aT, preferred_element_type=F32)
                      + b1a_row)
    anew_all = jnp.dot(hid, w1bT, preferred_element_type=F32) + b1b
    for k in range(CHUNK):
        g_col = gamma_ref[0, :, k:k + 1]
        r_is1 = r_ref[0, :, k:k + 1] == 1
        cond = (alpha - g_col) >= 0.0
        take_new = jnp.logical_not(jnp.logical_xor(r_is1, cond))
        alpha = jnp.where(take_new, anew_all[k * B:(k + 1) * B, :], alpha)
        alpha_out_ref[0, :, k:k + 1] = alpha

    hist_scr[...] = hist
    h_scr[...] = h
    alpha_scr[...] = alpha

    @pl.when(pid == NCHUNK - 1)
    def _drain():
        for k in range(CHUNK):
            t = pid * CHUNK + k
            pltpu.make_async_copy(
                slots.at[k], c3_out_ref.at[:, t, :], sems.at[k]).wait()


def _run_tc(gamma_c, r_c, c3_c, prev_c, WihT, WhhT, R_table, vd_row, vc3_row,
            bih_row, bhh_row, W1aT, w1b, b1a_row, b1b_2d,
            W2aT, w2b, b2a_row, b2b_2d, interpret=False):
    chunk_spec = pl.BlockSpec((1, B, CHUNK), lambda i: (i, 0, 0))

    def full(shape):
        nd = len(shape)
        return pl.BlockSpec(shape, lambda i, _n=nd: (0,) * _n)

    out_shapes = (
        jax.ShapeDtypeStruct((NCHUNK, B, CHUNK), F32),
        jax.ShapeDtypeStruct((B, S, DIM_V), F32),
        jax.ShapeDtypeStruct((B, S, NUM_C3), F32),
    )
    out_specs = (
        chunk_spec,
        pl.BlockSpec((B, CHUNK, DIM_V), lambda i: (0, i, 0)),
        pl.BlockSpec(memory_space=pl.ANY),
    )
    in_specs = [
        chunk_spec, chunk_spec, chunk_spec, chunk_spec,
        full(WihT.shape), full(WhhT.shape), full(R_table.shape),
        full(vd_row.shape), full(vc3_row.shape),
        full(bih_row.shape), full(bhh_row.shape),
        full(W1aT.shape), full(w1b.shape), full(b1a_row.shape),
        full(b1b_2d.shape),
        full(W2aT.shape), full(w2b.shape), full(b2a_row.shape),
        full(b2b_2d.shape),
    ]
    return pl.pallas_call(
        _tc_body,
        grid=(NCHUNK,),
        in_specs=in_specs,
        out_specs=out_specs,
        out_shape=out_shapes,
        scratch_shapes=[
            pltpu.VMEM((B, DIM_V), F32),
            pltpu.VMEM((B, 1), F32),
            pltpu.VMEM((B, HIST), F32),
            pltpu.VMEM((CHUNK, B, NUM_C3), F32),
            pltpu.SemaphoreType.DMA((CHUNK,)),
        ],
        compiler_params=pltpu.CompilerParams(
            dimension_semantics=("arbitrary",),
        ),
        interpret=interpret,
    )(gamma_c, r_c, c3_c, prev_c, WihT, WhhT, R_table, vd_row, vc3_row,
      bih_row, bhh_row, W1aT, w1b, b1a_row, b1b_2d,
      W2aT, w2b, b2a_row, b2b_2d)


def _chunked(x):
    return x.reshape(B, NCHUNK, CHUNK).transpose(1, 0, 2)


def kernel(c3_seq, d_seq, r_seq, v_c3, D_table, v_d, R_table, W_ih, W_hh,
           b_ih, b_hh, W1a, b1a, W1b, b1b, W2a, b2a, W2b, b2b):
    gamma_flat, prev_flat = _sc_prep(
        D_table.reshape(-1).astype(F32),
        d_seq.reshape(-1).astype(I32),
        c3_seq.reshape(-1).astype(I32),
    )
    gamma = gamma_flat.reshape(B, S)
    prev = prev_flat.reshape(B, S)

    alpha_c, h_seq, c3_out = _run_tc(
        _chunked(gamma),
        _chunked(r_seq.astype(I32)),
        _chunked(c3_seq.astype(I32)),
        _chunked(prev),
        W_ih.T, W_hh.T, R_table,
        v_d.reshape(1, DIM_V), v_c3.reshape(1, DIM_V),
        b_ih.reshape(1, 3 * DIM_V), b_hh.reshape(1, 3 * DIM_V),
        W1a.T, W1b.T, b1a.reshape(1, DIM_V), b1b.reshape(1, 1),
        W2a.T, W2b.T, b2a.reshape(1, DIM_V), b2b.reshape(1, 1),
    )
    alpha_seq = alpha_c.transpose(1, 0, 2).reshape(B, S)
    return alpha_seq, h_seq, c3_out


# speedup vs baseline: 1.0223x; 1.0223x over previous
import functools

import jax
import jax.numpy as jnp
from jax import lax
from jax.experimental import pallas as pl
from jax.experimental.pallas import tpu as pltpu
from jax.experimental.pallas import tpu_sc as plsc

B, S = 64, 200
NUM_C3, NUM_D, DIM_V = 2048, 10000, 128
CHUNK = 40
NCHUNK = S // CHUNK
HIST = 256
F32 = jnp.float32
I32 = jnp.int32


def _sc_prep(table_flat, d_flat, c3_flat):
    info = plsc.get_sparse_core_info()
    nc, ns, nl = info.num_cores, info.num_subcores, info.num_lanes
    nw = nc * ns
    total = B * S
    per = total // nw
    rows_per = per // S
    assert per % nl == 0 and rows_per * S == per

    mesh = plsc.VectorSubcoreMesh(core_axis_name="c", subcore_axis_name="s")

    @functools.partial(
        pl.kernel,
        mesh=mesh,
        out_type=(
            jax.ShapeDtypeStruct((total,), F32),
            jax.ShapeDtypeStruct((total,), I32),
        ),
        scratch_types=[
            pltpu.VMEM((NUM_D,), F32),
            pltpu.VMEM((per,), I32),
            pltpu.VMEM((per,), I32),
            pltpu.VMEM((per,), F32),
            pltpu.VMEM((per,), I32),
            pltpu.VMEM((rows_per * NUM_C3,), I32),
        ],
        compiler_params=pltpu.CompilerParams(needs_layout_passes=False),
    )
    def prep_kernel(tab_hbm, d_hbm, c3_hbm, gam_hbm, prev_hbm,
                    tab_v, d_v, c3_v, gam_v, prev_v, ls_v):
        wid = lax.axis_index("s") * nc + lax.axis_index("c")
        base = wid * per
        pltpu.sync_copy(tab_hbm, tab_v)
        pltpu.sync_copy(d_hbm.at[pl.ds(base, per)], d_v)
        pltpu.sync_copy(c3_hbm.at[pl.ds(base, per)], c3_v)

        for i in range(per // nl):
            sl = pl.ds(i * nl, nl)
            gam_v[sl] = plsc.load_gather(tab_v, [d_v[sl]])

        neg1 = jnp.full((nl,), -1, I32)
        for j in range(rows_per * NUM_C3 // nl):
            ls_v[pl.ds(j * nl, nl)] = neg1

        @pl.loop(0, S)
        def _chain(t):
            tv = jnp.zeros((nl,), I32) + t
            for r in range(rows_per):
                trow = tv + r * S
                idx = plsc.load_gather(c3_v, [trow]) + r * NUM_C3
                prev = plsc.load_gather(ls_v, [idx])
                plsc.store_scatter(ls_v, [idx], tv)
                plsc.store_scatter(prev_v, [trow], prev)

        pltpu.sync_copy(gam_v, gam_hbm.at[pl.ds(base, per)])
        pltpu.sync_copy(prev_v, prev_hbm.at[pl.ds(base, per)])

    return prep_kernel(table_flat, d_flat, c3_flat)


def _tc_body(gamma_ref, r_ref, c3_ref, prev_ref,
             WihT_ref, WhhT_ref, R_ref, vd_ref, vc3_ref,
             bih_ref, bhh_ref,
             W1aT_ref, w1b_ref, b1a_ref, b1b_ref,
             W2aT_ref, w2b_ref, b2a_ref, b2b_ref,
             alpha_out_ref, h_out_ref, c3_out_ref,
             h_scr, alpha_scr, hist_scr, slots, sems):
    pid = pl.program_id(0)

    @pl.when(pid == 0)
    def _init():
        h_scr[...] = jnp.zeros_like(h_scr)
        alpha_scr[...] = jnp.zeros_like(alpha_scr)
        hist_scr[...] = jnp.zeros_like(hist_scr)
        slots[CHUNK - 1] = jnp.zeros((B, NUM_C3), F32)

    vd_row = vd_ref[...]
    vc3_row = vc3_ref[...]

    bih_row = bih_ref[...]
    bhh_row = bhh_ref[...]
    b1a_row = b1a_ref[...]
    b2a_row = b2a_ref[...]
    w1bT = w1b_ref[...]
    w2bT = w2b_ref[...]
    b1b = b1b_ref[...]
    b2b = b2b_ref[...]
    WihT = WihT_ref[...]
    WhhT = WhhT_ref[...]
    W1aT = W1aT_ref[...]
    W2aT = W2aT_ref[...]
    R0 = R_ref[0:1, :]
    R1 = R_ref[1:2, :]

    iota_c3 = lax.broadcasted_iota(I32, (B, NUM_C3), 1)
    iota_h = lax.broadcasted_iota(I32, (B, HIST), 1)

    h = h_scr[...]
    alpha = alpha_scr[...]
    hist = hist_scr[...]

    xs = []
    for k in range(CHUNK):
        g_col = gamma_ref[0, :, k:k + 1]
        r_is1 = r_ref[0, :, k:k + 1] == 1
        vr_t = jnp.where(r_is1, R1, R0)
        xs.append(jnp.concatenate([g_col * vd_row, vr_t], axis=1))
    x_all = jnp.concatenate(xs, axis=0)
    gi_all = jnp.dot(x_all, WihT, preferred_element_type=F32) + bih_row

    hs = []
    for k in range(CHUNK):
        t = pid * CHUNK + k
        c_col = c3_ref[0, :, k:k + 1]
        p_col = prev_ref[0, :, k:k + 1]

        gi = gi_all[k * B:(k + 1) * B, :]
        gh = jnp.dot(h, WhhT, preferred_element_type=F32) + bhh_row
        rg = jax.nn.sigmoid(gi[:, 0:DIM_V] + gh[:, 0:DIM_V])
        zg = jax.nn.sigmoid(gi[:, DIM_V:2 * DIM_V] + gh[:, DIM_V:2 * DIM_V])
        ng = jnp.tanh(gi[:, 2 * DIM_V:] + rg * gh[:, 2 * DIM_V:])
        h = (1.0 - zg) * ng + zg * h
        h_out_ref[:, k, :] = h
        hs.append(h)

        bmask = iota_h == p_col
        beta3 = jnp.sum(jnp.where(bmask, hist, 0.0), axis=1, keepdims=True)
        cat = jnp.concatenate([beta3 * vc3_row,
                               x_all[k * B:(k + 1) * B, :]], axis=1)
        hid2 = jax.nn.relu(jnp.dot(cat, W2aT, preferred_element_type=F32)
                           + b2a_row)
        new_c3 = jnp.dot(hid2, w2bT, preferred_element_type=F32) + b2b
        hist = jnp.where(iota_h == t, new_c3, hist)

        @pl.when(pid > 0)
        def _wait_slot():
            pltpu.make_async_copy(
                slots.at[k], c3_out_ref.at[:, t, :], sems.at[k]).wait()
        prev_state = slots[k - 1] if k > 0 else slots[CHUNK - 1]
        slots[k] = jnp.where(iota_c3 == c_col, new_c3, prev_state)
        pltpu.make_async_copy(
            slots.at[k], c3_out_ref.at[:, t, :], sems.at[k]).start()

    h_all = jnp.concatenate(hs, axis=0)
    hid = jax.nn.relu(jnp.dot(h_all, W1aT, preferred_element_type=F32)
                      + b1a_row)
    anew_all = jnp.dot(hid, w1bT, preferred_element_type=F32) + b1b
    for k in range(CHUNK):
        g_col = gamma_ref[0, :, k:k + 1]
        r_is1 = r_ref[0, :, k:k + 1] == 1
        cond = (alpha - g_col) >= 0.0
        take_new = jnp.logical_not(jnp.logical_xor(r_is1, cond))
        alpha = jnp.where(take_new, anew_all[k * B:(k + 1) * B, :], alpha)
        alpha_out_ref[0, :, k:k + 1] = alpha

    hist_scr[...] = hist
    h_scr[...] = h
    alpha_scr[...] = alpha

    @pl.when(pid == NCHUNK - 1)
    def _drain():
        for k in range(CHUNK):
            t = pid * CHUNK + k
            pltpu.make_async_copy(
                slots.at[k], c3_out_ref.at[:, t, :], sems.at[k]).wait()


def _run_tc(gamma_c, r_c, c3_c, prev_c, WihT, WhhT, R_table, vd_row, vc3_row,
            bih_row, bhh_row, W1aT, w1b, b1a_row, b1b_2d,
            W2aT, w2b, b2a_row, b2b_2d, interpret=False):
    chunk_spec = pl.BlockSpec((1, B, CHUNK), lambda i: (i, 0, 0))

    def full(shape):
        nd = len(shape)
        return pl.BlockSpec(shape, lambda i, _n=nd: (0,) * _n)

    out_shapes = (
        jax.ShapeDtypeStruct((NCHUNK, B, CHUNK), F32),
        jax.ShapeDtypeStruct((B, S, DIM_V), F32),
        jax.ShapeDtypeStruct((B, S, NUM_C3), F32),
    )
    out_specs = (
        chunk_spec,
        pl.BlockSpec((B, CHUNK, DIM_V), lambda i: (0, i, 0)),
        pl.BlockSpec(memory_space=pl.ANY),
    )
    in_specs = [
        chunk_spec, chunk_spec, chunk_spec, chunk_spec,
        full(WihT.shape), full(WhhT.shape), full(R_table.shape),
        full(vd_row.shape), full(vc3_row.shape),
        full(bih_row.shape), full(bhh_row.shape),
        full(W1aT.shape), full(w1b.shape), full(b1a_row.shape),
        full(b1b_2d.shape),
        full(W2aT.shape), full(w2b.shape), full(b2a_row.shape),
        full(b2b_2d.shape),
    ]
    return pl.pallas_call(
        _tc_body,
        grid=(NCHUNK,),
        in_specs=in_specs,
        out_specs=out_specs,
        out_shape=out_shapes,
        scratch_shapes=[
            pltpu.VMEM((B, DIM_V), F32),
            pltpu.VMEM((B, 1), F32),
            pltpu.VMEM((B, HIST), F32),
            pltpu.VMEM((CHUNK, B, NUM_C3), F32),
            pltpu.SemaphoreType.DMA((CHUNK,)),
        ],
        compiler_params=pltpu.CompilerParams(
            dimension_semantics=("arbitrary",),
        ),
        interpret=interpret,
    )(gamma_c, r_c, c3_c, prev_c, WihT, WhhT, R_table, vd_row, vc3_row,
      bih_row, bhh_row, W1aT, w1b, b1a_row, b1b_2d,
      W2aT, w2b, b2a_row, b2b_2d)


def _chunked(x):
    return x.reshape(B, NCHUNK, CHUNK).transpose(1, 0, 2)


def kernel(c3_seq, d_seq, r_seq, v_c3, D_table, v_d, R_table, W_ih, W_hh,
           b_ih, b_hh, W1a, b1a, W1b, b1b, W2a, b2a, W2b, b2b):
    gamma_flat, prev_flat = _sc_prep(
        D_table.reshape(-1).astype(F32),
        d_seq.reshape(-1).astype(I32),
        c3_seq.reshape(-1).astype(I32),
    )
    gamma = gamma_flat.reshape(B, S)
    prev = prev_flat.reshape(B, S)

    alpha_c, h_seq, c3_out = _run_tc(
        _chunked(gamma),
        _chunked(r_seq.astype(I32)),
        _chunked(c3_seq.astype(I32)),
        _chunked(prev),
        W_ih.T, W_hh.T, R_table,
        v_d.reshape(1, DIM_V), v_c3.reshape(1, DIM_V),
        b_ih.reshape(1, 3 * DIM_V), b_hh.reshape(1, 3 * DIM_V),
        W1a.T, W1b.T, b1a.reshape(1, DIM_V), b1b.reshape(1, 1),
        W2a.T, W2b.T, b2a.reshape(1, DIM_V), b2b.reshape(1, 1),
    )
    alpha_seq = alpha_c.transpose(1, 0, 2).reshape(B, S)
    return alpha_seq, h_seq, c3_out
